# Initial kernel scaffold; baseline (speedup 1.0000x reference)
import jax
import jax.numpy as jnp
from jax.experimental import pallas as pl


def kernel(x, edge_index, batch, emb, gin_W1, gin_b1, gin_W2, gin_b2, gin_eps, agg_seed, agg_Wk, agg_Wv, agg_Wo, st_seed, st_Wk, st_Wv, st_Wo):
    raise NotImplementedError("write your pallas kernel here")


# speedup vs baseline: 3.5633x; 3.5633x over previous
import functools

import jax
import jax.numpy as jnp
from jax import lax
from jax.experimental import pallas as pl
from jax.experimental.pallas import tpu as pltpu
from jax.experimental.pallas import tpu_sc as plsc

DIM = 128
N_LAYER = 4
HEADS_LAYER = 2
HEADS_FINAL = 4
N_NODES = 10000
N_EDGES = 320000
NUM_GRAPHS = 128

E_TOT = 2 * N_EDGES
NW = 32
EDGE_CHUNK = 128
CHUNKS_PER_W = 160
E_PAD = NW * CHUNKS_PER_W * EDGE_CHUNK
ROWS_PAD = 10240
ROWS_PER_TILE = ROWS_PAD // 16
TRASH = N_NODES


def _sc_aggregate(h, srcm, dstm, zrows):
    mesh = plsc.VectorSubcoreMesh(core_axis_name="c", subcore_axis_name="s")

    @functools.partial(
        pl.kernel,
        out_type=jax.ShapeDtypeStruct((2, ROWS_PAD, DIM), jnp.float32),
        mesh=mesh,
        scratch_types=[
            pltpu.VMEM((EDGE_CHUNK,), jnp.int32),
            pltpu.VMEM((EDGE_CHUNK,), jnp.int32),
            pltpu.VMEM((EDGE_CHUNK, DIM), jnp.float32),
            pltpu.VMEM_SHARED((ROWS_PAD, DIM), jnp.float32),
            pltpu.SemaphoreType.DMA,
        ],
    )
    def body(h_hbm, src_hbm, dst_hbm, z_hbm, out_hbm,
             src_v, dst_v, rows_v, accum, sem):
        c = lax.axis_index("c")
        s = lax.axis_index("s")
        w = c * 16 + s
        pltpu.sync_copy(z_hbm, accum.at[pl.ds(s * ROWS_PER_TILE, ROWS_PER_TILE)])
        plsc.subcore_barrier()

        def step(b, carry):
            pltpu.sync_copy(src_hbm.at[w, b], src_v)
            pltpu.sync_copy(dst_hbm.at[w, b], dst_v)
            pltpu.async_copy(h_hbm.at[src_v], rows_v, sem).wait()
            pltpu.sync_copy(rows_v, accum.at[dst_v], add=True)
            return carry

        lax.fori_loop(0, CHUNKS_PER_W, step, 0)
        plsc.subcore_barrier()
        pltpu.sync_copy(
            accum.at[pl.ds(s * ROWS_PER_TILE, ROWS_PER_TILE)],
            out_hbm.at[c, pl.ds(s * ROWS_PER_TILE, ROWS_PER_TILE)])

    return body(h, srcm, dstm, zrows)


def _dot3(a, b):
    ah = a.astype(jnp.bfloat16)
    al = (a - ah.astype(jnp.float32)).astype(jnp.bfloat16)
    bh = b.astype(jnp.bfloat16)
    bl = (b - bh.astype(jnp.float32)).astype(jnp.bfloat16)
    f = jnp.float32
    return (jnp.dot(ah, bh, preferred_element_type=f)
            + jnp.dot(ah, bl, preferred_element_type=f)
            + jnp.dot(al, bh, preferred_element_type=f))


_EMB_BLK = 1000


def _embed_body(x_ref, emb_ref, out_ref):
    x = x_ref[...]
    h = jnp.zeros((_EMB_BLK, DIM), jnp.float32)
    for i in range(16):
        h = jnp.where(x == i, emb_ref[i, :][None, :], h)
    out_ref[...] = h


def _embed(x2d, emb):
    return pl.pallas_call(
        _embed_body,
        grid=(N_NODES // _EMB_BLK,),
        in_specs=[
            pl.BlockSpec((_EMB_BLK, 1), lambda i: (i, 0)),
            pl.BlockSpec((16, DIM), lambda i: (0, 0)),
        ],
        out_specs=pl.BlockSpec((_EMB_BLK, DIM), lambda i: (i, 0)),
        out_shape=jax.ShapeDtypeStruct((N_NODES, DIM), jnp.float32),
    )(x2d, emb)


def _layer_body(h_ref, agg_ref, eps_ref, batch_ref,
                W1_ref, b1_ref, W2_ref, b2_ref,
                Wk_ref, Wv_ref, Wo_ref, seed_ref,
                hout_ref, rep_ref):
    h = h_ref[...]
    aggr = agg_ref[0, :N_NODES, :] + agg_ref[1, :N_NODES, :]
    z = eps_ref[...] * h + aggr
    z = _dot3(z, W1_ref[...]) + b1_ref[...]
    z = jnp.maximum(z, 0.0)
    z = _dot3(z, W2_ref[...]) + b2_ref[...]
    hn = h + z
    hout_ref[...] = hn

    k = jnp.dot(hn, Wk_ref[...], preferred_element_type=jnp.float32)
    v = jnp.dot(hn, Wv_ref[...], preferred_element_type=jnp.float32)
    kq = k * seed_ref[...]
    scale = 1.0 / (64.0 ** 0.5)
    s0 = jnp.sum(kq[:, :64], axis=1, keepdims=True) * scale
    s1 = jnp.sum(kq[:, 64:], axis=1, keepdims=True) * scale
    onehot = (batch_ref[...] ==
              lax.broadcasted_iota(jnp.int32, (N_NODES, NUM_GRAPHS), 1))
    ohf = onehot.astype(jnp.float32)
    neg = jnp.float32(-1e30)
    sm0 = jnp.max(jnp.where(onehot, s0, neg), axis=0, keepdims=True)
    sm1 = jnp.max(jnp.where(onehot, s1, neg), axis=0, keepdims=True)
    smax = jnp.concatenate([sm0, sm1], axis=0)
    sm_n = jnp.dot(ohf, smax.T, preferred_element_type=jnp.float32)
    e0 = jnp.exp(s0 - sm_n[:, 0:1])
    e1 = jnp.exp(s1 - sm_n[:, 1:2])
    e = jnp.concatenate([e0, e1], axis=1)
    denom = lax.dot_general(ohf, e, (((0,), (0,)), ((), ())),
                            preferred_element_type=jnp.float32)
    den_n = jnp.dot(ohf, denom, preferred_element_type=jnp.float32)
    a0 = e0 / den_n[:, 0:1]
    a1 = e1 / den_n[:, 1:2]
    wv = jnp.concatenate([v[:, :64] * a0, v[:, 64:] * a1], axis=1)
    outg = lax.dot_general(ohf, wv, (((0,), (0,)), ((), ())),
                           preferred_element_type=jnp.float32)
    rep_ref[...] = jnp.dot(outg, Wo_ref[...], preferred_element_type=jnp.float32)


def _layer_tc(h, agg, one_plus_eps, batch2d, W1, b1, W2, b2, Wk, Wv, Wo, seed):
    return pl.pallas_call(
        _layer_body,
        out_shape=[
            jax.ShapeDtypeStruct((N_NODES, DIM), jnp.float32),
            jax.ShapeDtypeStruct((NUM_GRAPHS, DIM), jnp.float32),
        ],
    )(h, agg, one_plus_eps, batch2d, W1, b1, W2, b2, Wk, Wv, Wo, seed)


def _final_body(reps_ref, Wk_ref, Wv_ref, Wo_ref, seed_ref, out_ref):
    scale = 1.0 / (32.0 ** 0.5)
    ks = []
    vs = []
    for l in range(N_LAYER):
        r = reps_ref[l]
        ks.append(jnp.dot(r, Wk_ref[...], preferred_element_type=jnp.float32))
        vs.append(jnp.dot(r, Wv_ref[...], preferred_element_type=jnp.float32))
    scores = []
    for l in range(N_LAYER):
        kq = ks[l] * seed_ref[...]
        cols = [jnp.sum(kq[:, 32 * t:32 * (t + 1)], axis=1, keepdims=True) * scale
                for t in range(HEADS_FINAL)]
        scores.append(jnp.concatenate(cols, axis=1))
    smax = scores[0]
    for l in range(1, N_LAYER):
        smax = jnp.maximum(smax, scores[l])
    es = [jnp.exp(scores[l] - smax) for l in range(N_LAYER)]
    denom = es[0]
    for l in range(1, N_LAYER):
        denom = denom + es[l]
    out = jnp.zeros((NUM_GRAPHS, DIM), jnp.float32)
    for l in range(N_LAYER):
        a = es[l] / denom
        pieces = [vs[l][:, 32 * t:32 * (t + 1)] * a[:, t:t + 1]
                  for t in range(HEADS_FINAL)]
        out = out + jnp.concatenate(pieces, axis=1)
    out_ref[...] = jnp.dot(out, Wo_ref[...], preferred_element_type=jnp.float32)


def _final_tc(reps, Wk, Wv, Wo, seed):
    return pl.pallas_call(
        _final_body,
        out_shape=jax.ShapeDtypeStruct((NUM_GRAPHS, DIM), jnp.float32),
    )(reps, Wk, Wv, Wo, seed)


def kernel(x, edge_index, batch, emb, gin_W1, gin_b1, gin_W2, gin_b2, gin_eps,
           agg_seed, agg_Wk, agg_Wv, agg_Wo, st_seed, st_Wk, st_Wv, st_Wo):
    s0 = edge_index[0].astype(jnp.int32)
    d0 = edge_index[1].astype(jnp.int32)
    eid = jnp.concatenate([s0 * N_NODES + d0, d0 * N_NODES + s0])
    eid = jnp.sort(eid)
    dup = jnp.concatenate(
        [jnp.zeros((1,), bool), eid[1:] == eid[:-1]])
    src = eid // N_NODES
    dst = eid % N_NODES
    valid = (src != dst) & ~dup
    src = jnp.where(valid, src, 0)
    dst = jnp.where(valid, dst, TRASH)
    pad = E_PAD - E_TOT
    src = jnp.concatenate([src, jnp.zeros((pad,), jnp.int32)])
    dst = jnp.concatenate([dst, jnp.full((pad,), TRASH, jnp.int32)])
    srcm = src.reshape(NW, CHUNKS_PER_W, EDGE_CHUNK)
    dstm = dst.reshape(NW, CHUNKS_PER_W, EDGE_CHUNK)
    zrows = jnp.zeros((ROWS_PER_TILE, DIM), jnp.float32)

    x2d = x.astype(jnp.int32).reshape(N_NODES, 1)
    batch2d = batch.astype(jnp.int32).reshape(N_NODES, 1)

    h = _embed(x2d, emb)
    reps = []
    for l in range(N_LAYER):
        agg = _sc_aggregate(h, srcm, dstm, zrows)
        ope = jnp.broadcast_to(1.0 + gin_eps[l], (1, DIM)).astype(jnp.float32)
        h, rep = _layer_tc(h, agg, ope, batch2d,
                           gin_W1[l], gin_b1[l].reshape(1, DIM),
                           gin_W2[l], gin_b2[l].reshape(1, DIM),
                           agg_Wk[l], agg_Wv[l], agg_Wo[l],
                           agg_seed[l].reshape(1, DIM))
        reps.append(rep)
    reps = jnp.stack(reps)
    return _final_tc(reps, st_Wk, st_Wv, st_Wo, st_seed.reshape(1, DIM))


# speedup vs baseline: 3.9704x; 1.1142x over previous
import functools

import jax
import jax.numpy as jnp
from jax import lax
from jax.experimental import pallas as pl
from jax.experimental.pallas import tpu as pltpu
from jax.experimental.pallas import tpu_sc as plsc

DIM = 128
N_LAYER = 4
HEADS_LAYER = 2
HEADS_FINAL = 4
N_NODES = 10000
N_EDGES = 320000
NUM_GRAPHS = 128

E_TOT = 2 * N_EDGES
NW = 32
EDGE_CHUNK = 128
CHUNKS_PER_W = 160
E_PAD = NW * CHUNKS_PER_W * EDGE_CHUNK
ROWS_PAD = 10240
ROWS_PER_TILE = ROWS_PAD // 16
TRASH = N_NODES


def _sc_aggregate(h, sd, zrows):
    mesh = plsc.VectorSubcoreMesh(core_axis_name="c", subcore_axis_name="s")

    @functools.partial(
        pl.kernel,
        out_type=jax.ShapeDtypeStruct((2, ROWS_PAD, DIM), jnp.float32),
        mesh=mesh,
        scratch_types=[
            pltpu.VMEM((2, EDGE_CHUNK), jnp.int32),
            pltpu.VMEM((2, EDGE_CHUNK), jnp.int32),
            pltpu.VMEM((EDGE_CHUNK, DIM), jnp.float32),
            pltpu.VMEM((EDGE_CHUNK, DIM), jnp.float32),
            pltpu.VMEM_SHARED((ROWS_PAD, DIM), jnp.float32),
            pltpu.SemaphoreType.DMA,
            pltpu.SemaphoreType.DMA,
        ],
    )
    def body(h_hbm, sd_hbm, z_hbm, out_hbm,
             sdA, sdB, rows0, rows1, accum, sem0, sem1):
        c = lax.axis_index("c")
        s = lax.axis_index("s")
        w = c * 16 + s
        pltpu.sync_copy(z_hbm, accum.at[pl.ds(s * ROWS_PER_TILE, ROWS_PER_TILE)])
        plsc.subcore_barrier()
        pltpu.sync_copy(sd_hbm.at[w, 0], sdA)
        pltpu.sync_copy(sd_hbm.at[w, 1], sdB)
        pltpu.async_copy(h_hbm.at[sdA.at[0]], rows0, sem0)
        half = CHUNKS_PER_W // 2

        def step(i, carry):
            pltpu.async_copy(h_hbm.at[sdB.at[0]], rows1, sem1)
            pltpu.make_async_copy(h_hbm.at[sdA.at[0]], rows0, sem0).wait()
            pltpu.sync_copy(rows0, accum.at[sdA.at[1]], add=True)

            @pl.when(i < half - 1)
            def _():
                pltpu.sync_copy(sd_hbm.at[w, 2 * i + 2], sdA)
                pltpu.async_copy(h_hbm.at[sdA.at[0]], rows0, sem0)

            pltpu.make_async_copy(h_hbm.at[sdB.at[0]], rows1, sem1).wait()
            pltpu.sync_copy(rows1, accum.at[sdB.at[1]], add=True)

            @pl.when(i < half - 1)
            def _():
                pltpu.sync_copy(sd_hbm.at[w, 2 * i + 3], sdB)

            return carry

        lax.fori_loop(0, half, step, 0)
        plsc.subcore_barrier()
        pltpu.sync_copy(
            accum.at[pl.ds(s * ROWS_PER_TILE, ROWS_PER_TILE)],
            out_hbm.at[c, pl.ds(s * ROWS_PER_TILE, ROWS_PER_TILE)])

    return body(h, sd, zrows)


def _oh_dotT(ohf, m):
    f = jnp.float32
    dims = (((0,), (0,)), ((), ()))
    ob = ohf.astype(jnp.bfloat16)
    mh = m.astype(jnp.bfloat16)
    ml = (m - mh.astype(f)).astype(jnp.bfloat16)
    return (lax.dot_general(ob, mh, dims, preferred_element_type=f)
            + lax.dot_general(ob, ml, dims, preferred_element_type=f))


def _vdotT(m, ohf):
    f = jnp.float32
    dims = (((0,), (0,)), ((), ()))
    ob = ohf.astype(jnp.bfloat16)
    mh = m.astype(jnp.bfloat16)
    ml = (m - mh.astype(f)).astype(jnp.bfloat16)
    return (lax.dot_general(mh, ob, dims, preferred_element_type=f)
            + lax.dot_general(ml, ob, dims, preferred_element_type=f))


_EMB_BLK = 1000


def _embed_body(x_ref, emb_ref, out_ref):
    x = x_ref[...]
    h = jnp.zeros((_EMB_BLK, DIM), jnp.float32)
    for i in range(16):
        h = jnp.where(x == i, emb_ref[i, :][None, :], h)
    out_ref[...] = h


def _embed(x2d, emb):
    return pl.pallas_call(
        _embed_body,
        grid=(N_NODES // _EMB_BLK,),
        in_specs=[
            pl.BlockSpec((_EMB_BLK, 1), lambda i: (i, 0)),
            pl.BlockSpec((16, DIM), lambda i: (0, 0)),
        ],
        out_specs=pl.BlockSpec((_EMB_BLK, DIM), lambda i: (i, 0)),
        out_shape=jax.ShapeDtypeStruct((N_NODES, DIM), jnp.float32),
    )(x2d, emb)


def _layer_body(h_ref, agg_ref, eps_ref, batch_ref,
                W1_ref, b1_ref, W2_ref, b2_ref,
                Wk_ref, Wv_ref, Wo_ref, seed_ref,
                hout_ref, rep_ref):
    h = h_ref[...]
    aggr = agg_ref[0, :N_NODES, :] + agg_ref[1, :N_NODES, :]
    z = eps_ref[...] * h + aggr
    z = jnp.dot(z, W1_ref[...], preferred_element_type=jnp.float32) + b1_ref[...]
    z = jnp.maximum(z, 0.0)
    z = jnp.dot(z, W2_ref[...], preferred_element_type=jnp.float32) + b2_ref[...]
    hn = h + z
    hout_ref[...] = hn

    k = jnp.dot(hn, Wk_ref[...], preferred_element_type=jnp.float32)
    v = jnp.dot(hn, Wv_ref[...], preferred_element_type=jnp.float32)
    row = lax.broadcasted_iota(jnp.int32, (DIM, HEADS_LAYER), 0)
    col = lax.broadcasted_iota(jnp.int32, (DIM, HEADS_LAYER), 1)
    qb = jnp.where(row // 64 == col, seed_ref[...], 0.0)
    scale = 1.0 / (64.0 ** 0.5)
    sc = jnp.dot(k, qb, preferred_element_type=jnp.float32) * scale
    s0 = sc[:, 0:1]
    s1 = sc[:, 1:2]
    onehot = (batch_ref[...] ==
              lax.broadcasted_iota(jnp.int32, (N_NODES, NUM_GRAPHS), 1))
    ohf = onehot.astype(jnp.float32)
    neg = jnp.float32(-1e30)
    sm0 = jnp.max(jnp.where(onehot, s0, neg), axis=0, keepdims=True)
    sm1 = jnp.max(jnp.where(onehot, s1, neg), axis=0, keepdims=True)
    smn0 = jnp.max(jnp.where(onehot, sm0, neg), axis=1, keepdims=True)
    smn1 = jnp.max(jnp.where(onehot, sm1, neg), axis=1, keepdims=True)
    e0 = jnp.exp(s0 - smn0)
    e1 = jnp.exp(s1 - smn1)
    den0 = _vdotT(e0, ohf)
    den1 = _vdotT(e1, ohf)
    dn0 = jnp.max(jnp.where(onehot, den0, neg), axis=1, keepdims=True)
    dn1 = jnp.max(jnp.where(onehot, den1, neg), axis=1, keepdims=True)
    a0 = e0 / dn0
    a1 = e1 / dn1
    wv = jnp.concatenate([v[:, :64] * a0, v[:, 64:] * a1], axis=1)
    outg = _oh_dotT(ohf, wv)
    rep_ref[...] = jnp.dot(outg, Wo_ref[...], preferred_element_type=jnp.float32)


def _layer_tc(h, agg, one_plus_eps, batch2d, W1, b1, W2, b2, Wk, Wv, Wo, seed):
    return pl.pallas_call(
        _layer_body,
        out_shape=[
            jax.ShapeDtypeStruct((N_NODES, DIM), jnp.float32),
            jax.ShapeDtypeStruct((NUM_GRAPHS, DIM), jnp.float32),
        ],
    )(h, agg, one_plus_eps, batch2d, W1, b1, W2, b2, Wk, Wv, Wo, seed)


def _final_body(reps_ref, Wk_ref, Wv_ref, Wo_ref, seed_ref, out_ref):
    scale = 1.0 / (32.0 ** 0.5)
    ks = []
    vs = []
    for l in range(N_LAYER):
        r = reps_ref[l]
        ks.append(jnp.dot(r, Wk_ref[...], preferred_element_type=jnp.float32))
        vs.append(jnp.dot(r, Wv_ref[...], preferred_element_type=jnp.float32))
    row = lax.broadcasted_iota(jnp.int32, (DIM, HEADS_FINAL), 0)
    col = lax.broadcasted_iota(jnp.int32, (DIM, HEADS_FINAL), 1)
    qb = jnp.where(row // 32 == col, seed_ref[...], 0.0)
    scores = [jnp.dot(ks[l], qb, preferred_element_type=jnp.float32) * scale
              for l in range(N_LAYER)]
    smax = scores[0]
    for l in range(1, N_LAYER):
        smax = jnp.maximum(smax, scores[l])
    es = [jnp.exp(scores[l] - smax) for l in range(N_LAYER)]
    denom = es[0]
    for l in range(1, N_LAYER):
        denom = denom + es[l]
    out = jnp.zeros((NUM_GRAPHS, DIM), jnp.float32)
    for l in range(N_LAYER):
        a = es[l] / denom
        pieces = [vs[l][:, 32 * t:32 * (t + 1)] * a[:, t:t + 1]
                  for t in range(HEADS_FINAL)]
        out = out + jnp.concatenate(pieces, axis=1)
    out_ref[...] = jnp.dot(out, Wo_ref[...], preferred_element_type=jnp.float32)


def _final_tc(reps, Wk, Wv, Wo, seed):
    return pl.pallas_call(
        _final_body,
        out_shape=jax.ShapeDtypeStruct((NUM_GRAPHS, DIM), jnp.float32),
    )(reps, Wk, Wv, Wo, seed)


def kernel(x, edge_index, batch, emb, gin_W1, gin_b1, gin_W2, gin_b2, gin_eps,
           agg_seed, agg_Wk, agg_Wv, agg_Wo, st_seed, st_Wk, st_Wv, st_Wo):
    s0 = edge_index[0].astype(jnp.int32)
    d0 = edge_index[1].astype(jnp.int32)
    eid = jnp.concatenate([s0 * N_NODES + d0, d0 * N_NODES + s0])
    eid = jnp.sort(eid)
    dup = jnp.concatenate(
        [jnp.zeros((1,), bool), eid[1:] == eid[:-1]])
    src = eid // N_NODES
    dst = eid % N_NODES
    valid = (src != dst) & ~dup
    sentinel = N_NODES * N_NODES
    order = jnp.argsort(jnp.where(valid, dst * N_NODES + src, sentinel))
    src = src[order]
    dst = dst[order]
    valid = valid[order]
    cnt = jnp.sum(valid.astype(jnp.int32))
    wslots = CHUNKS_PER_W * EDGE_CHUNK
    nominal = (jnp.arange(1, NW, dtype=jnp.int32) * cnt) // NW
    rows_at = dst[nominal]
    snapped = jnp.searchsorted(dst, rows_at, side="left").astype(jnp.int32)
    snapped = jnp.clip(snapped, nominal - 450, nominal + 450)
    starts = jnp.concatenate([jnp.zeros((1,), jnp.int32), snapped])
    ends = jnp.concatenate([snapped, cnt.reshape(1)])
    nw_n = ends - starts
    slot = jnp.arange(wslots, dtype=jnp.int32)[None, :]
    gidx = jnp.clip(starts[:, None] + slot, 0, E_TOT - 1)
    live = slot < nw_n[:, None]
    srcm = jnp.where(live, src[gidx], 0)
    dstm = jnp.where(live, dst[gidx], TRASH)
    srcm = srcm.reshape(NW, CHUNKS_PER_W, 1, EDGE_CHUNK)
    dstm = dstm.reshape(NW, CHUNKS_PER_W, 1, EDGE_CHUNK)
    sd = jnp.concatenate([srcm, dstm], axis=2)
    zrows = jnp.zeros((ROWS_PER_TILE, DIM), jnp.float32)

    x2d = x.astype(jnp.int32).reshape(N_NODES, 1)
    batch2d = batch.astype(jnp.int32).reshape(N_NODES, 1)

    h = _embed(x2d, emb)
    reps = []
    for l in range(N_LAYER):
        agg = _sc_aggregate(h, sd, zrows)
        ope = jnp.broadcast_to(1.0 + gin_eps[l], (1, DIM)).astype(jnp.float32)
        h, rep = _layer_tc(h, agg, ope, batch2d,
                           gin_W1[l], gin_b1[l].reshape(1, DIM),
                           gin_W2[l], gin_b2[l].reshape(1, DIM),
                           agg_Wk[l], agg_Wv[l], agg_Wo[l],
                           agg_seed[l].reshape(DIM, 1))
        reps.append(rep)
    reps = jnp.stack(reps)
    return _final_tc(reps, st_Wk, st_Wv, st_Wo, st_seed.reshape(DIM, 1))
